# baseline (speedup 1.0000x reference)
import jax
import jax.numpy as jnp
from jax.experimental import pallas as pl
from jax.experimental.pallas import tpu as pltpu
from jax.experimental.pallas import tpu_sc as plsc

_B, _L, _H, _E, _R, _TOP_K = 4, 2048, 1024, 8, 64, 2
_KR = _TOP_K * _R
_LT = 2048
_LANES = 16


def _lane_gather(v, perm):
    dnums = jax.lax.GatherDimensionNumbers(
        offset_dims=(), collapsed_slice_dims=(0,), start_index_map=(0,))
    return jax.lax.gather(
        v, perm[:, None], dnums, slice_sizes=(1,),
        mode=jax.lax.GatherScatterMode.PROMISE_IN_BOUNDS)


def _sc_router_body(cls_hbm, rw_hbm, idx_hbm, gates_hbm,
                    cls_v, rw_v, val_v, row_v):
    c = jax.lax.axis_index("c")
    s = jax.lax.axis_index("s")

    @pl.when(jnp.logical_and(c == 0, s < _B))
    def _():
        b = s
        pltpu.sync_copy(cls_hbm.at[b], cls_v)
        pltpu.sync_copy(rw_hbm, rw_v)
        lanes = jax.lax.iota(jnp.int32, _LANES)
        logits = jnp.full((_LANES,), -jnp.inf, jnp.float32)
        for e in range(_E):
            def dot_step(j, acc, e=e):
                return (acc
                        + cls_v[pl.ds(j * _LANES, _LANES)]
                        * rw_v[e, pl.ds(j * _LANES, _LANES)])
            acc = jax.lax.fori_loop(0, _H // _LANES, dot_step,
                                    jnp.zeros((_LANES,), jnp.float32))
            for sh in (8, 4, 2, 1):
                perm = jnp.bitwise_xor(lanes, sh)
                acc = acc + _lane_gather(acc, perm)
            logits = jnp.where(lanes == e, acc, logits)
        def allmax(v):
            for sh in (8, 4, 2, 1):
                v = jnp.maximum(v, _lane_gather(v, jnp.bitwise_xor(lanes, sh)))
            return v

        def allmin(v):
            for sh in (8, 4, 2, 1):
                v = jnp.minimum(v, _lane_gather(v, jnp.bitwise_xor(lanes, sh)))
            return v

        m1 = allmax(logits)
        i1 = allmin(jnp.where(logits == m1, lanes, _LANES))
        rest = jnp.where(lanes == i1, -jnp.inf, logits)
        m2 = allmax(rest)
        i2 = allmin(jnp.where(rest == m2, lanes, _LANES))
        val_v[...] = jnp.where(lanes == 0, i1, i2)
        g1 = 1.0 / (1.0 + jnp.exp(m2 - m1))
        row_v[...] = jnp.where(lanes == 0, g1, 1.0 - g1)
        pltpu.sync_copy(val_v, idx_hbm.at[b])
        pltpu.sync_copy(row_v, gates_hbm.at[b])


def _mix_body(idx_ref, gates_ref, a0_ref, a1_ref, b0_ref, b1_ref,
              x_ref, o_ref):
    b = pl.program_id(0) // (_L // _LT)
    srow = jax.lax.broadcasted_iota(jnp.int32, (_TOP_K, _KR), 0)
    scol = jax.lax.broadcasted_iota(jnp.int32, (_TOP_K, _KR), 1)
    sel = (scol // _R == srow).astype(jnp.float32)
    gv = jnp.dot(gates_ref[pl.ds(b, 1), :], sel,
                 preferred_element_type=jnp.float32)

    a2 = jnp.concatenate([a0_ref[0], a1_ref[0]], axis=1)
    bcat = jnp.concatenate([b0_ref[0], b1_ref[0]], axis=0)
    xb = x_ref[0].astype(jnp.bfloat16)
    low = jnp.dot(xb, a2, preferred_element_type=jnp.float32)
    low = (low * gv).astype(jnp.bfloat16)
    up = jnp.dot(low, bcat, preferred_element_type=jnp.float32)
    o_ref[0] = x_ref[0] + up


def kernel(x, router_w, lora_a, lora_b):
    cls = x[:, 0, :]
    router = pl.kernel(
        _sc_router_body,
        out_type=(jax.ShapeDtypeStruct((_B, _LANES), jnp.int32),
                  jax.ShapeDtypeStruct((_B, _LANES), jnp.float32)),
        mesh=plsc.VectorSubcoreMesh(core_axis_name="c", subcore_axis_name="s"),
        scratch_types=[
            pltpu.VMEM((_H,), jnp.float32),
            pltpu.VMEM((_E, _H), jnp.float32),
            pltpu.VMEM((_LANES,), jnp.int32),
            pltpu.VMEM((_LANES,), jnp.float32),
        ],
    )
    idx16, g16 = router(cls, router_w)
    idx = idx16[:, :_TOP_K]
    gates = g16[:, :_TOP_K]

    a16 = lora_a.astype(jnp.bfloat16)
    b16 = lora_b.astype(jnp.bfloat16)
    grid_spec = pltpu.PrefetchScalarGridSpec(
        num_scalar_prefetch=1,
        grid=(_B * _L // _LT,),
        in_specs=[
            pl.BlockSpec((_B, _TOP_K), lambda t, i: (0, 0)),
            pl.BlockSpec((1, _H, _R),
                         lambda t, i: (i[t // (_L // _LT), 0], 0, 0)),
            pl.BlockSpec((1, _H, _R),
                         lambda t, i: (i[t // (_L // _LT), 1], 0, 0)),
            pl.BlockSpec((1, _R, _H),
                         lambda t, i: (i[t // (_L // _LT), 0], 0, 0)),
            pl.BlockSpec((1, _R, _H),
                         lambda t, i: (i[t // (_L // _LT), 1], 0, 0)),
            pl.BlockSpec((1, _LT, _H),
                         lambda t, i: (t // (_L // _LT), t % (_L // _LT), 0)),
        ],
        out_specs=pl.BlockSpec((1, _LT, _H),
                               lambda t, i: (t // (_L // _LT), t % (_L // _LT), 0)),
    )
    return pl.pallas_call(
        _mix_body,
        grid_spec=grid_spec,
        out_shape=jax.ShapeDtypeStruct((_B, _L, _H), jnp.float32),
    )(idx, gates, a16, a16, b16, b16, x)


# speedup vs baseline: 1.0165x; 1.0165x over previous
import jax
import jax.numpy as jnp
from jax.experimental import pallas as pl
from jax.experimental.pallas import tpu as pltpu
from jax.experimental.pallas import tpu_sc as plsc

_B, _L, _H, _E, _R, _TOP_K = 4, 2048, 1024, 8, 64, 2
_KR = _TOP_K * _R
_LT = 2048
_LANES = 16


def _lane_gather(v, perm):
    dnums = jax.lax.GatherDimensionNumbers(
        offset_dims=(), collapsed_slice_dims=(0,), start_index_map=(0,))
    return jax.lax.gather(
        v, perm[:, None], dnums, slice_sizes=(1,),
        mode=jax.lax.GatherScatterMode.PROMISE_IN_BOUNDS)


def _sc_router_body(cls_hbm, rw_hbm, idx_hbm, gates_hbm,
                    cls_v, rw_v, val_v, row_v):
    c = jax.lax.axis_index("c")
    s = jax.lax.axis_index("s")

    @pl.when(jnp.logical_and(c == 0, s < _B))
    def _():
        b = s
        pltpu.sync_copy(cls_hbm.at[b], cls_v)
        pltpu.sync_copy(rw_hbm, rw_v)
        lanes = jax.lax.iota(jnp.int32, _LANES)
        logits = jnp.full((_LANES,), -jnp.inf, jnp.float32)
        zero = jnp.zeros((_LANES,), jnp.float32)
        for e in range(_E):
            def dot_step(j, accs, e=e):
                a0, a1 = accs
                base = j * 2 * _LANES
                return (a0 + cls_v[pl.ds(base, _LANES)]
                        * rw_v[e, pl.ds(base, _LANES)],
                        a1 + cls_v[pl.ds(base + _LANES, _LANES)]
                        * rw_v[e, pl.ds(base + _LANES, _LANES)])
            a0, a1 = jax.lax.fori_loop(0, _H // (2 * _LANES), dot_step,
                                       (zero, zero), unroll=4)
            acc = a0 + a1
            for sh in (8, 4, 2, 1):
                perm = jnp.bitwise_xor(lanes, sh)
                acc = acc + _lane_gather(acc, perm)
            logits = jnp.where(lanes == e, acc, logits)
        def allmax(v):
            for sh in (8, 4, 2, 1):
                v = jnp.maximum(v, _lane_gather(v, jnp.bitwise_xor(lanes, sh)))
            return v

        def allmin(v):
            for sh in (8, 4, 2, 1):
                v = jnp.minimum(v, _lane_gather(v, jnp.bitwise_xor(lanes, sh)))
            return v

        m1 = allmax(logits)
        i1 = allmin(jnp.where(logits == m1, lanes, _LANES))
        rest = jnp.where(lanes == i1, -jnp.inf, logits)
        m2 = allmax(rest)
        i2 = allmin(jnp.where(rest == m2, lanes, _LANES))
        val_v[...] = jnp.where(lanes == 0, i1, i2)
        g1 = 1.0 / (1.0 + jnp.exp(m2 - m1))
        row_v[...] = jnp.where(lanes == 0, g1, 1.0 - g1)
        pltpu.sync_copy(val_v, idx_hbm.at[b])
        pltpu.sync_copy(row_v, gates_hbm.at[b])


def _mix_body(idx_ref, gates_ref, a0_ref, a1_ref, b0_ref, b1_ref,
              x_ref, o_ref):
    b = pl.program_id(0) // (_L // _LT)
    srow = jax.lax.broadcasted_iota(jnp.int32, (_TOP_K, _KR), 0)
    scol = jax.lax.broadcasted_iota(jnp.int32, (_TOP_K, _KR), 1)
    sel = (scol // _R == srow).astype(jnp.float32)
    gv = jnp.dot(gates_ref[pl.ds(b, 1), :], sel,
                 preferred_element_type=jnp.float32)

    a2 = jnp.concatenate([a0_ref[0], a1_ref[0]], axis=1)
    bcat = jnp.concatenate([b0_ref[0], b1_ref[0]], axis=0)
    xb = x_ref[0].astype(jnp.bfloat16)
    low = jnp.dot(xb, a2, preferred_element_type=jnp.float32)
    low = (low * gv).astype(jnp.bfloat16)
    up = jnp.dot(low, bcat, preferred_element_type=jnp.float32)
    o_ref[0] = x_ref[0] + up


def kernel(x, router_w, lora_a, lora_b):
    cls = x[:, 0, :]
    router = pl.kernel(
        _sc_router_body,
        out_type=(jax.ShapeDtypeStruct((_B, _LANES), jnp.int32),
                  jax.ShapeDtypeStruct((_B, _LANES), jnp.float32)),
        mesh=plsc.VectorSubcoreMesh(core_axis_name="c", subcore_axis_name="s"),
        scratch_types=[
            pltpu.VMEM((_H,), jnp.float32),
            pltpu.VMEM((_E, _H), jnp.float32),
            pltpu.VMEM((_LANES,), jnp.int32),
            pltpu.VMEM((_LANES,), jnp.float32),
        ],
    )
    idx16, g16 = router(cls, router_w)
    idx = idx16[:, :_TOP_K]
    gates = g16[:, :_TOP_K]

    a16 = lora_a.astype(jnp.bfloat16)
    b16 = lora_b.astype(jnp.bfloat16)
    grid_spec = pltpu.PrefetchScalarGridSpec(
        num_scalar_prefetch=1,
        grid=(_B * _L // _LT,),
        in_specs=[
            pl.BlockSpec((_B, _TOP_K), lambda t, i: (0, 0)),
            pl.BlockSpec((1, _H, _R),
                         lambda t, i: (i[t // (_L // _LT), 0], 0, 0)),
            pl.BlockSpec((1, _H, _R),
                         lambda t, i: (i[t // (_L // _LT), 1], 0, 0)),
            pl.BlockSpec((1, _R, _H),
                         lambda t, i: (i[t // (_L // _LT), 0], 0, 0)),
            pl.BlockSpec((1, _R, _H),
                         lambda t, i: (i[t // (_L // _LT), 1], 0, 0)),
            pl.BlockSpec((1, _LT, _H),
                         lambda t, i: (t // (_L // _LT), t % (_L // _LT), 0)),
        ],
        out_specs=pl.BlockSpec((1, _LT, _H),
                               lambda t, i: (t // (_L // _LT), t % (_L // _LT), 0)),
    )
    return pl.pallas_call(
        _mix_body,
        grid_spec=grid_spec,
        out_shape=jax.ShapeDtypeStruct((_B, _L, _H), jnp.float32),
    )(idx, gates, a16, a16, b16, b16, x)


# speedup vs baseline: 1.0481x; 1.0311x over previous
import jax
import jax.numpy as jnp
from jax.experimental import pallas as pl
from jax.experimental.pallas import tpu as pltpu
from jax.experimental.pallas import tpu_sc as plsc

_B, _L, _H, _E, _R, _TOP_K = 4, 2048, 1024, 8, 64, 2
_KR = _TOP_K * _R
_LT = 2048
_LANES = 16


def _lane_gather(v, perm):
    dnums = jax.lax.GatherDimensionNumbers(
        offset_dims=(), collapsed_slice_dims=(0,), start_index_map=(0,))
    return jax.lax.gather(
        v, perm[:, None], dnums, slice_sizes=(1,),
        mode=jax.lax.GatherScatterMode.PROMISE_IN_BOUNDS)


def _sc_router_body(cls_hbm, rw_hbm, idx_hbm, gates_hbm,
                    cls_v, rw_v, val_v, row_v):
    c = jax.lax.axis_index("c")
    s = jax.lax.axis_index("s")

    @pl.when(jnp.logical_and(c == 0, s < _B))
    def _():
        b = s
        pltpu.sync_copy(cls_hbm.at[b], cls_v)
        pltpu.sync_copy(rw_hbm, rw_v)
        lanes = jax.lax.iota(jnp.int32, _LANES)
        logits = jnp.full((_LANES,), -jnp.inf, jnp.float32)
        zero = jnp.zeros((_LANES,), jnp.float32)
        for e in range(_E):
            def dot_step(j, accs, e=e):
                a0, a1 = accs
                base = j * 2 * _LANES
                return (a0 + cls_v[pl.ds(base, _LANES)]
                        * rw_v[e, pl.ds(base, _LANES)],
                        a1 + cls_v[pl.ds(base + _LANES, _LANES)]
                        * rw_v[e, pl.ds(base + _LANES, _LANES)])
            a0, a1 = jax.lax.fori_loop(0, _H // (2 * _LANES), dot_step,
                                       (zero, zero), unroll=4)
            acc = a0 + a1
            for sh in (8, 4, 2, 1):
                perm = jnp.bitwise_xor(lanes, sh)
                acc = acc + _lane_gather(acc, perm)
            logits = jnp.where(lanes == e, acc, logits)
        def allmax(v):
            for sh in (8, 4, 2, 1):
                v = jnp.maximum(v, _lane_gather(v, jnp.bitwise_xor(lanes, sh)))
            return v

        def allmin(v):
            for sh in (8, 4, 2, 1):
                v = jnp.minimum(v, _lane_gather(v, jnp.bitwise_xor(lanes, sh)))
            return v

        m1 = allmax(logits)
        i1 = allmin(jnp.where(logits == m1, lanes, _LANES))
        rest = jnp.where(lanes == i1, -jnp.inf, logits)
        m2 = allmax(rest)
        i2 = allmin(jnp.where(rest == m2, lanes, _LANES))
        val_v[...] = jnp.where(lanes == 0, i1, i2)
        g1 = 1.0 / (1.0 + jnp.exp(m2 - m1))
        row_v[...] = jnp.where(lanes == 0, g1, 1.0 - g1)
        pltpu.sync_copy(val_v, idx_hbm.at[b])
        pltpu.sync_copy(row_v, gates_hbm.at[b])


def _mix_body(idx_ref, gates_ref, a0_ref, a1_ref, b0_ref, b1_ref,
              x_ref, o_ref):
    b = pl.program_id(0) // (_L // _LT)
    srow = jax.lax.broadcasted_iota(jnp.int32, (_TOP_K, _KR), 0)
    scol = jax.lax.broadcasted_iota(jnp.int32, (_TOP_K, _KR), 1)
    sel = (scol // _R == srow).astype(jnp.float32)
    gv = jnp.dot(gates_ref[pl.ds(b, 1), :], sel,
                 preferred_element_type=jnp.float32)

    a2 = jnp.concatenate([a0_ref[0], a1_ref[0]], axis=1)
    bcat = jnp.concatenate([b0_ref[0], b1_ref[0]], axis=0)
    xb = x_ref[0].astype(jnp.bfloat16)
    low = jnp.dot(xb, a2, preferred_element_type=jnp.float32)
    low = (low * gv).astype(jnp.bfloat16)
    up = jnp.dot(low, bcat, preferred_element_type=jnp.float32)
    o_ref[0] = x_ref[0] + up


def kernel(x, router_w, lora_a, lora_b):
    cls = x[:, 0, :]
    router = pl.kernel(
        _sc_router_body,
        out_type=(jax.ShapeDtypeStruct((_B, _LANES), jnp.int32),
                  jax.ShapeDtypeStruct((_B, _LANES), jnp.float32)),
        mesh=plsc.VectorSubcoreMesh(core_axis_name="c", subcore_axis_name="s",
                                    num_cores=1),
        scratch_types=[
            pltpu.VMEM((_H,), jnp.float32),
            pltpu.VMEM((_E, _H), jnp.float32),
            pltpu.VMEM((_LANES,), jnp.int32),
            pltpu.VMEM((_LANES,), jnp.float32),
        ],
    )
    idx16, g16 = router(cls, router_w)
    idx = idx16[:, :_TOP_K]
    gates = g16[:, :_TOP_K]

    a16 = lora_a.astype(jnp.bfloat16)
    b16 = lora_b.astype(jnp.bfloat16)
    grid_spec = pltpu.PrefetchScalarGridSpec(
        num_scalar_prefetch=1,
        grid=(_B * _L // _LT,),
        in_specs=[
            pl.BlockSpec((_B, _TOP_K), lambda t, i: (0, 0)),
            pl.BlockSpec((1, _H, _R),
                         lambda t, i: (i[t // (_L // _LT), 0], 0, 0)),
            pl.BlockSpec((1, _H, _R),
                         lambda t, i: (i[t // (_L // _LT), 1], 0, 0)),
            pl.BlockSpec((1, _R, _H),
                         lambda t, i: (i[t // (_L // _LT), 0], 0, 0)),
            pl.BlockSpec((1, _R, _H),
                         lambda t, i: (i[t // (_L // _LT), 1], 0, 0)),
            pl.BlockSpec((1, _LT, _H),
                         lambda t, i: (t // (_L // _LT), t % (_L // _LT), 0)),
        ],
        out_specs=pl.BlockSpec((1, _LT, _H),
                               lambda t, i: (t // (_L // _LT), t % (_L // _LT), 0)),
    )
    return pl.pallas_call(
        _mix_body,
        grid_spec=grid_spec,
        out_shape=jax.ShapeDtypeStruct((_B, _L, _H), jnp.float32),
    )(idx, gates, a16, a16, b16, b16, x)


# speedup vs baseline: 1.0515x; 1.0032x over previous
import jax
import jax.numpy as jnp
from jax.experimental import pallas as pl
from jax.experimental.pallas import tpu as pltpu
from jax.experimental.pallas import tpu_sc as plsc

_B, _L, _H, _E, _R, _TOP_K = 4, 2048, 1024, 8, 64, 2
_KR = _TOP_K * _R
_LT = 2048
_LANES = 16


def _lane_gather(v, perm):
    dnums = jax.lax.GatherDimensionNumbers(
        offset_dims=(), collapsed_slice_dims=(0,), start_index_map=(0,))
    return jax.lax.gather(
        v, perm[:, None], dnums, slice_sizes=(1,),
        mode=jax.lax.GatherScatterMode.PROMISE_IN_BOUNDS)


def _sc_router_body(cls_hbm, rw_hbm, out_hbm,
                    cls_v, rw_v, pack_v, sem1, sem2):
    c = jax.lax.axis_index("c")
    s = jax.lax.axis_index("s")

    @pl.when(jnp.logical_and(c == 0, s < _B))
    def _():
        b = s
        cp1 = pltpu.make_async_copy(cls_hbm.at[b], cls_v, sem1)
        cp2 = pltpu.make_async_copy(rw_hbm, rw_v, sem2)
        cp1.start()
        cp2.start()
        cp1.wait()
        cp2.wait()
        lanes = jax.lax.iota(jnp.int32, _LANES)
        logits = jnp.full((_LANES,), -jnp.inf, jnp.float32)
        zero = jnp.zeros((_LANES,), jnp.float32)

        def dot_step(j, accs):
            base = j * _LANES
            cv = cls_v[pl.ds(base, _LANES)]
            return tuple(a + cv * rw_v[e, pl.ds(base, _LANES)]
                         for e, a in enumerate(accs))
        accs = jax.lax.fori_loop(0, _H // _LANES, dot_step,
                                 (zero,) * _E, unroll=2)
        for e in range(_E):
            acc = accs[e]
            for sh in (8, 4, 2, 1):
                perm = jnp.bitwise_xor(lanes, sh)
                acc = acc + _lane_gather(acc, perm)
            logits = jnp.where(lanes == e, acc, logits)
        def allmax(v):
            for sh in (8, 4, 2, 1):
                v = jnp.maximum(v, _lane_gather(v, jnp.bitwise_xor(lanes, sh)))
            return v

        def allmin(v):
            for sh in (8, 4, 2, 1):
                v = jnp.minimum(v, _lane_gather(v, jnp.bitwise_xor(lanes, sh)))
            return v

        m1 = allmax(logits)
        i1 = allmin(jnp.where(logits == m1, lanes, _LANES))
        rest = jnp.where(lanes == i1, -jnp.inf, logits)
        m2 = allmax(rest)
        i2 = allmin(jnp.where(rest == m2, lanes, _LANES))
        g1 = 1.0 / (1.0 + jnp.exp(m2 - m1))
        gates_row = jnp.where(lanes == 0, g1, 1.0 - g1)
        pack_v[pl.ds(0, _LANES)] = jnp.where(lanes == 0, i1, i2).astype(
            jnp.float32)
        pack_v[pl.ds(_LANES, _LANES)] = gates_row
        pltpu.sync_copy(pack_v, out_hbm.at[b])


def _mix_body(idx_ref, gates_ref, a0_ref, a1_ref, b0_ref, b1_ref,
              x_ref, o_ref):
    b = pl.program_id(0) // (_L // _LT)
    srow = jax.lax.broadcasted_iota(jnp.int32, (_TOP_K, _KR), 0)
    scol = jax.lax.broadcasted_iota(jnp.int32, (_TOP_K, _KR), 1)
    sel = (scol // _R == srow).astype(jnp.float32)
    gv = jnp.dot(gates_ref[pl.ds(b, 1), :], sel,
                 preferred_element_type=jnp.float32)

    a2 = jnp.concatenate([a0_ref[0], a1_ref[0]], axis=1)
    bcat = jnp.concatenate([b0_ref[0], b1_ref[0]], axis=0)
    xb = x_ref[0].astype(jnp.bfloat16)
    low = jnp.dot(xb, a2, preferred_element_type=jnp.float32)
    low = (low * gv).astype(jnp.bfloat16)
    up = jnp.dot(low, bcat, preferred_element_type=jnp.float32)
    o_ref[0] = x_ref[0] + up


def kernel(x, router_w, lora_a, lora_b):
    cls = x[:, 0, :]
    router = pl.kernel(
        _sc_router_body,
        out_type=jax.ShapeDtypeStruct((_B, 2 * _LANES), jnp.float32),
        mesh=plsc.VectorSubcoreMesh(core_axis_name="c", subcore_axis_name="s",
                                    num_cores=1),
        scratch_types=[
            pltpu.VMEM((_H,), jnp.float32),
            pltpu.VMEM((_E, _H), jnp.float32),
            pltpu.VMEM((2 * _LANES,), jnp.float32),
            pltpu.SemaphoreType.DMA,
            pltpu.SemaphoreType.DMA,
        ],
    )
    packed = router(cls, router_w)
    idx = packed[:, :_TOP_K].astype(jnp.int32)
    gates = packed[:, _LANES:_LANES + _TOP_K]

    a16 = lora_a.astype(jnp.bfloat16)
    b16 = lora_b.astype(jnp.bfloat16)
    grid_spec = pltpu.PrefetchScalarGridSpec(
        num_scalar_prefetch=1,
        grid=(_B * _L // _LT,),
        in_specs=[
            pl.BlockSpec((_B, _TOP_K), lambda t, i: (0, 0)),
            pl.BlockSpec((1, _H, _R),
                         lambda t, i: (i[t // (_L // _LT), 0], 0, 0)),
            pl.BlockSpec((1, _H, _R),
                         lambda t, i: (i[t // (_L // _LT), 1], 0, 0)),
            pl.BlockSpec((1, _R, _H),
                         lambda t, i: (i[t // (_L // _LT), 0], 0, 0)),
            pl.BlockSpec((1, _R, _H),
                         lambda t, i: (i[t // (_L // _LT), 1], 0, 0)),
            pl.BlockSpec((1, _LT, _H),
                         lambda t, i: (t // (_L // _LT), t % (_L // _LT), 0)),
        ],
        out_specs=pl.BlockSpec((1, _LT, _H),
                               lambda t, i: (t // (_L // _LT), t % (_L // _LT), 0)),
    )
    return pl.pallas_call(
        _mix_body,
        grid_spec=grid_spec,
        out_shape=jax.ShapeDtypeStruct((_B, _L, _H), jnp.float32),
    )(idx, gates, a16, a16, b16, b16, x)


# speedup vs baseline: 1.0841x; 1.0309x over previous
import jax
import jax.numpy as jnp
from jax.experimental import pallas as pl
from jax.experimental.pallas import tpu as pltpu
from jax.experimental.pallas import tpu_sc as plsc

_B, _L, _H, _E, _R, _TOP_K = 4, 2048, 1024, 8, 64, 2
_KR = _TOP_K * _R
_LT = 2048
_LANES = 16


def _lane_gather(v, perm):
    dnums = jax.lax.GatherDimensionNumbers(
        offset_dims=(), collapsed_slice_dims=(0,), start_index_map=(0,))
    return jax.lax.gather(
        v, perm[:, None], dnums, slice_sizes=(1,),
        mode=jax.lax.GatherScatterMode.PROMISE_IN_BOUNDS)


def _sc_router_body(cls_hbm, rw_hbm, idx_hbm, gates_hbm,
                    cls_v, rw_v, idx_v, row_v, sem1, sem2):
    c = jax.lax.axis_index("c")
    s = jax.lax.axis_index("s")

    @pl.when(jnp.logical_and(c == 0, s < _B))
    def _():
        b = s
        cp1 = pltpu.make_async_copy(cls_hbm.at[b], cls_v, sem1)
        cp2 = pltpu.make_async_copy(rw_hbm, rw_v, sem2)
        cp1.start()
        cp2.start()
        cp1.wait()
        cp2.wait()
        lanes = jax.lax.iota(jnp.int32, _LANES)
        logits = jnp.full((_LANES,), -jnp.inf, jnp.float32)
        zero = jnp.zeros((_LANES,), jnp.float32)

        def dot_step(j, accs):
            base = j * _LANES
            cv = cls_v[pl.ds(base, _LANES)]
            return tuple(a + cv * rw_v[e, pl.ds(base, _LANES)]
                         for e, a in enumerate(accs))
        accs = jax.lax.fori_loop(0, _H // _LANES, dot_step,
                                 (zero,) * _E, unroll=2)
        for e in range(_E):
            acc = accs[e]
            for sh in (8, 4, 2, 1):
                perm = jnp.bitwise_xor(lanes, sh)
                acc = acc + _lane_gather(acc, perm)
            logits = jnp.where(lanes == e, acc, logits)
        def allmax(v):
            for sh in (8, 4, 2, 1):
                v = jnp.maximum(v, _lane_gather(v, jnp.bitwise_xor(lanes, sh)))
            return v

        def allmin(v):
            for sh in (8, 4, 2, 1):
                v = jnp.minimum(v, _lane_gather(v, jnp.bitwise_xor(lanes, sh)))
            return v

        m1 = allmax(logits)
        i1 = allmin(jnp.where(logits == m1, lanes, _LANES))
        rest = jnp.where(lanes == i1, -jnp.inf, logits)
        m2 = allmax(rest)
        i2 = allmin(jnp.where(rest == m2, lanes, _LANES))
        g1 = 1.0 / (1.0 + jnp.exp(m2 - m1))
        gates_row = jnp.where(lanes == 0, g1, 1.0 - g1)
        idx_v[...] = jnp.where(lanes == 0, i1, i2)
        row_v[...] = gates_row
        pltpu.sync_copy(idx_v, idx_hbm.at[b])
        pltpu.sync_copy(row_v, gates_hbm.at[b])


def _mix_body(idx_ref, gates_ref, a0_ref, a1_ref, b0_ref, b1_ref,
              x_ref, o_ref):
    b = pl.program_id(0) // (_L // _LT)
    srow = jax.lax.broadcasted_iota(jnp.int32, (_LANES, _KR), 0)
    scol = jax.lax.broadcasted_iota(jnp.int32, (_LANES, _KR), 1)
    sel = (scol // _R == srow).astype(jnp.float32)
    gv = jnp.dot(gates_ref[pl.ds(b, 1), :], sel,
                 preferred_element_type=jnp.float32)

    a2 = jnp.concatenate([a0_ref[0], a1_ref[0]], axis=1)
    bcat = jnp.concatenate([b0_ref[0], b1_ref[0]], axis=0)
    xb = x_ref[0].astype(jnp.bfloat16)
    low = jnp.dot(xb, a2, preferred_element_type=jnp.float32)
    low = (low * gv).astype(jnp.bfloat16)
    up = jnp.dot(low, bcat, preferred_element_type=jnp.float32)
    o_ref[0] = x_ref[0] + up


def kernel(x, router_w, lora_a, lora_b):
    cls = x[:, 0, :]
    router = pl.kernel(
        _sc_router_body,
        out_type=(jax.ShapeDtypeStruct((_B, _LANES), jnp.int32),
                  jax.ShapeDtypeStruct((_B, _LANES), jnp.float32)),
        mesh=plsc.VectorSubcoreMesh(core_axis_name="c", subcore_axis_name="s",
                                    num_cores=1),
        scratch_types=[
            pltpu.VMEM((_H,), jnp.float32),
            pltpu.VMEM((_E, _H), jnp.float32),
            pltpu.VMEM((_LANES,), jnp.int32),
            pltpu.VMEM((_LANES,), jnp.float32),
            pltpu.SemaphoreType.DMA,
            pltpu.SemaphoreType.DMA,
        ],
    )
    idx16, g16 = router(cls, router_w)

    a16 = lora_a.astype(jnp.bfloat16)
    b16 = lora_b.astype(jnp.bfloat16)
    grid_spec = pltpu.PrefetchScalarGridSpec(
        num_scalar_prefetch=1,
        grid=(_B * _L // _LT,),
        in_specs=[
            pl.BlockSpec((_B, _LANES), lambda t, i: (0, 0)),
            pl.BlockSpec((1, _H, _R),
                         lambda t, i: (i[t // (_L // _LT), 0], 0, 0)),
            pl.BlockSpec((1, _H, _R),
                         lambda t, i: (i[t // (_L // _LT), 1], 0, 0)),
            pl.BlockSpec((1, _R, _H),
                         lambda t, i: (i[t // (_L // _LT), 0], 0, 0)),
            pl.BlockSpec((1, _R, _H),
                         lambda t, i: (i[t // (_L // _LT), 1], 0, 0)),
            pl.BlockSpec((1, _LT, _H),
                         lambda t, i: (t // (_L // _LT), t % (_L // _LT), 0)),
        ],
        out_specs=pl.BlockSpec((1, _LT, _H),
                               lambda t, i: (t // (_L // _LT), t % (_L // _LT), 0)),
    )
    return pl.pallas_call(
        _mix_body,
        grid_spec=grid_spec,
        out_shape=jax.ShapeDtypeStruct((_B, _L, _H), jnp.float32),
    )(idx16, g16, a16, a16, b16, b16, x)
